# Initial kernel scaffold; baseline (speedup 1.0000x reference)
import jax
import jax.numpy as jnp
from jax.experimental import pallas as pl


def kernel(input, expert_sel, keys, values):
    raise NotImplementedError("write your pallas kernel here")


# speedup vs baseline: 1.6538x; 1.6538x over previous
import functools
import math

import jax
import jax.numpy as jnp
from jax.experimental import pallas as pl
from jax.experimental.pallas import tpu as pltpu

D_MODEL = 768
N_EXPERTS = 64
EXPERT_SIZE = 64
TOP_K = 8
BT = 256


def _moe_kernel(x_ref, selt_ref, k2_ref, v2_ref, rep_ref, o_ref):
    x = x_ref[...]
    xb = x.astype(jnp.bfloat16)
    logits = jax.lax.dot_general(
        xb, selt_ref[...].astype(jnp.bfloat16), (((1,), (0,)), ((), ())),
        preferred_element_type=jnp.float32)
    sel = jax.nn.sigmoid(logits)

    col = jax.lax.broadcasted_iota(jnp.int32, sel.shape, 1)
    work = sel
    mask = jnp.zeros(sel.shape, jnp.float32)
    for _ in range(TOP_K):
        m = jnp.max(work, axis=1, keepdims=True)
        eq = work == m
        amin = jnp.min(jnp.where(eq, col, N_EXPERTS), axis=1, keepdims=True)
        first = col == amin
        mask = jnp.where(first, 1.0, mask)
        work = jnp.where(first, -1.0, work)
    w = sel * mask

    wide_w = jax.lax.dot_general(
        w, rep_ref[...], (((1,), (0,)), ((), ())),
        precision=jax.lax.Precision.HIGHEST,
        preferred_element_type=jnp.float32)

    scores = jax.lax.dot_general(
        xb, k2_ref[...], (((1,), (0,)), ((), ())),
        preferred_element_type=jnp.float32)
    z = jnp.maximum(scores, 0.0) * wide_w
    o_ref[...] = jax.lax.dot_general(
        z.astype(jnp.bfloat16), v2_ref[...], (((1,), (0,)), ((), ())),
        preferred_element_type=jnp.float32)


@jax.jit
def kernel(input, expert_sel, keys, values):
    n_tokens = input.shape[0]
    selt = expert_sel.T
    k2 = keys.astype(jnp.bfloat16).transpose(1, 0, 2).reshape(D_MODEL, N_EXPERTS * EXPERT_SIZE)
    v2 = values.astype(jnp.bfloat16).reshape(N_EXPERTS * EXPERT_SIZE, D_MODEL)
    rep = jnp.repeat(jnp.eye(N_EXPERTS, dtype=jnp.float32), EXPERT_SIZE, axis=1)

    grid = (n_tokens // BT,)
    out = pl.pallas_call(
        _moe_kernel,
        grid=grid,
        in_specs=[
            pl.BlockSpec((BT, D_MODEL), lambda i: (i, 0)),
            pl.BlockSpec((D_MODEL, N_EXPERTS), lambda i: (0, 0)),
            pl.BlockSpec((D_MODEL, N_EXPERTS * EXPERT_SIZE), lambda i: (0, 0)),
            pl.BlockSpec((N_EXPERTS * EXPERT_SIZE, D_MODEL), lambda i: (0, 0)),
            pl.BlockSpec((N_EXPERTS, N_EXPERTS * EXPERT_SIZE), lambda i: (0, 0)),
        ],
        out_specs=pl.BlockSpec((BT, D_MODEL), lambda i: (i, 0)),
        out_shape=jax.ShapeDtypeStruct((n_tokens, D_MODEL), jnp.float32),
    )(input, selt, k2, v2, rep)
    return out


# speedup vs baseline: 1.8789x; 1.1361x over previous
import functools
import math

import jax
import jax.numpy as jnp
from jax.experimental import pallas as pl
from jax.experimental.pallas import tpu as pltpu

D_MODEL = 768
N_EXPERTS = 64
EXPERT_SIZE = 64
TOP_K = 8
BT = 256


def _moe_kernel(x_ref, selt_ref, k2_ref, v2_ref, o_ref):
    x = x_ref[...]
    xb = x.astype(jnp.bfloat16)
    logits = jax.lax.dot_general(
        xb, selt_ref[...], (((1,), (0,)), ((), ())),
        preferred_element_type=jnp.float32)
    sel = jax.nn.sigmoid(logits)

    col = jax.lax.broadcasted_iota(jnp.int32, sel.shape, 1)
    work = sel
    mask = jnp.zeros(sel.shape, jnp.float32)
    for _ in range(TOP_K):
        m = jnp.max(work, axis=1, keepdims=True)
        eq = work == m
        amin = jnp.min(jnp.where(eq, col, N_EXPERTS), axis=1, keepdims=True)
        first = col == amin
        mask = jnp.where(first, 1.0, mask)
        work = jnp.where(first, -1.0, work)
    w = sel * mask

    wide_w = pltpu.repeat(w, EXPERT_SIZE, axis=1)

    scores = jax.lax.dot_general(
        xb, k2_ref[...], (((1,), (0,)), ((), ())),
        preferred_element_type=jnp.float32)
    z = jnp.maximum(scores, 0.0) * wide_w
    o_ref[...] = jax.lax.dot_general(
        z.astype(jnp.bfloat16), v2_ref[...], (((1,), (0,)), ((), ())),
        preferred_element_type=jnp.float32)


@jax.jit
def kernel(input, expert_sel, keys, values):
    n_tokens = input.shape[0]
    selt = expert_sel.T.astype(jnp.bfloat16)
    k2 = keys.astype(jnp.bfloat16).transpose(1, 2, 0).reshape(
        D_MODEL, N_EXPERTS * EXPERT_SIZE)
    v2 = values.astype(jnp.bfloat16).transpose(1, 0, 2).reshape(
        N_EXPERTS * EXPERT_SIZE, D_MODEL)

    grid = (n_tokens // BT,)
    out = pl.pallas_call(
        _moe_kernel,
        grid=grid,
        in_specs=[
            pl.BlockSpec((BT, D_MODEL), lambda i: (i, 0)),
            pl.BlockSpec((D_MODEL, N_EXPERTS), lambda i: (0, 0)),
            pl.BlockSpec((D_MODEL, N_EXPERTS * EXPERT_SIZE), lambda i: (0, 0)),
            pl.BlockSpec((N_EXPERTS * EXPERT_SIZE, D_MODEL), lambda i: (0, 0)),
        ],
        out_specs=pl.BlockSpec((BT, D_MODEL), lambda i: (i, 0)),
        out_shape=jax.ShapeDtypeStruct((n_tokens, D_MODEL), jnp.float32),
    )(input, selt, k2, v2)
    return out


# speedup vs baseline: 2.0628x; 1.0979x over previous
import functools
import math

import jax
import jax.numpy as jnp
from jax.experimental import pallas as pl
from jax.experimental.pallas import tpu as pltpu

D_MODEL = 768
N_EXPERTS = 64
EXPERT_SIZE = 64
TOP_K = 8
BT = 256


def _moe_kernel(x_ref, selt_ref, k2_ref, v2_ref, rv_ref, o_ref):
    x = x_ref[...]
    xb = x.astype(jnp.bfloat16)
    logits = jax.lax.dot_general(
        xb, selt_ref[...], (((1,), (0,)), ((), ())),
        preferred_element_type=jnp.float32)
    sel = jax.nn.sigmoid(logits)

    col = jax.lax.broadcasted_iota(jnp.int32, sel.shape, 1)
    work = sel
    mask = jnp.zeros(sel.shape, jnp.float32)
    for _ in range(TOP_K):
        m = jnp.max(work, axis=1, keepdims=True)
        eq = work == m
        amin = jnp.min(jnp.where(eq, col, N_EXPERTS), axis=1, keepdims=True)
        first = col == amin
        mask = jnp.where(first, 1.0, mask)
        work = jnp.where(first, -1.0, work)
    w = (sel * mask).astype(jnp.bfloat16)

    wide_w = jax.lax.dot_general(
        w, rv_ref[...], (((1,), (0,)), ((), ())),
        preferred_element_type=jnp.float32)

    scores = jax.lax.dot_general(
        xb, k2_ref[...], (((1,), (0,)), ((), ())),
        preferred_element_type=jnp.float32)
    z = jnp.maximum(scores, 0.0) * wide_w
    o_ref[...] = jax.lax.dot_general(
        z.astype(jnp.bfloat16), v2_ref[...], (((1,), (0,)), ((), ())),
        preferred_element_type=jnp.float32)


@jax.jit
def kernel(input, expert_sel, keys, values):
    n_tokens = input.shape[0]
    selt = expert_sel.T.astype(jnp.bfloat16)
    k2 = keys.astype(jnp.bfloat16).transpose(1, 0, 2).reshape(
        D_MODEL, N_EXPERTS * EXPERT_SIZE)
    v2 = values.astype(jnp.bfloat16).reshape(N_EXPERTS * EXPERT_SIZE, D_MODEL)
    rv = jnp.repeat(jnp.eye(N_EXPERTS, dtype=jnp.bfloat16), EXPERT_SIZE, axis=1)

    grid = (n_tokens // BT,)
    out = pl.pallas_call(
        _moe_kernel,
        grid=grid,
        in_specs=[
            pl.BlockSpec((BT, D_MODEL), lambda i: (i, 0)),
            pl.BlockSpec((D_MODEL, N_EXPERTS), lambda i: (0, 0)),
            pl.BlockSpec((D_MODEL, N_EXPERTS * EXPERT_SIZE), lambda i: (0, 0)),
            pl.BlockSpec((N_EXPERTS * EXPERT_SIZE, D_MODEL), lambda i: (0, 0)),
            pl.BlockSpec((N_EXPERTS, N_EXPERTS * EXPERT_SIZE), lambda i: (0, 0)),
        ],
        out_specs=pl.BlockSpec((BT, D_MODEL), lambda i: (i, 0)),
        out_shape=jax.ShapeDtypeStruct((n_tokens, D_MODEL), jnp.float32),
    )(input, selt, k2, v2, rv)
    return out
